# baseline (speedup 1.0000x reference)
import jax
import jax.numpy as jnp
from jax.experimental import pallas as pl
from jax.experimental.pallas import tpu as pltpu

_BM = 512
_NBUF = 4


def _gate_kernel(x_hbm, w_ref, o_hbm):
    m, k = x_hbm.shape
    e = w_ref.shape[0]

    def body(x_blk, o_blk):
        o_blk[...] = jax.lax.dot_general(
            x_blk[...], w_ref[...], (((1,), (1,)), ((), ())),
            precision=jax.lax.Precision.DEFAULT,
            preferred_element_type=jnp.float32)

    pltpu.emit_pipeline(
        body,
        grid=(m // _BM,),
        in_specs=[pl.BlockSpec((_BM, k), lambda i: (i, 0),
                               pipeline_mode=pl.Buffered(buffer_count=_NBUF))],
        out_specs=[pl.BlockSpec((_BM, e), lambda i: (i, 0))],
    )(x_hbm, o_hbm)


def kernel(hidden_states, weight):
    m, k = hidden_states.shape
    e = weight.shape[0]
    return pl.pallas_call(
        _gate_kernel,
        in_specs=[
            pl.BlockSpec(memory_space=pltpu.HBM),
            pl.BlockSpec(memory_space=pltpu.VMEM),
        ],
        out_specs=pl.BlockSpec(memory_space=pltpu.HBM),
        out_shape=jax.ShapeDtypeStruct((m, e), jnp.float32),
        compiler_params=pltpu.CompilerParams(
            disable_bounds_checks=True,
            skip_device_barrier=True),
    )(hidden_states, weight)


# speedup vs baseline: 1.0451x; 1.0451x over previous
import jax
import jax.numpy as jnp
from jax.experimental import pallas as pl
from jax.experimental.pallas import tpu as pltpu

_BM = 512


def _gate_kernel(x_ref, w_ref, o_ref):
    o_ref[...] = jax.lax.dot_general(
        x_ref[...], w_ref[...], (((1,), (1,)), ((), ())),
        precision=jax.lax.Precision.DEFAULT,
        preferred_element_type=jnp.float32)


def kernel(hidden_states, weight):
    m, k = hidden_states.shape
    e = weight.shape[0]
    return pl.pallas_call(
        _gate_kernel,
        grid=(m // _BM,),
        in_specs=[
            pl.BlockSpec((_BM, k), lambda i: (i, 0)),
            pl.BlockSpec((e, k), lambda i: (0, 0)),
        ],
        out_specs=pl.BlockSpec((_BM, e), lambda i: (i, 0)),
        out_shape=jax.ShapeDtypeStruct((m, e), jnp.float32),
        compiler_params=pltpu.CompilerParams(
            dimension_semantics=(pltpu.PARALLEL,),
            disable_bounds_checks=True,
            skip_device_barrier=True),
    )(hidden_states, weight)
